# baseline (speedup 1.0000x reference)
import functools

import jax
import jax.numpy as jnp
from jax import lax
from jax.experimental import pallas as pl
from jax.experimental.pallas import tpu as pltpu
from jax.experimental.pallas import tpu_sc as plsc

B = 16384
S = 50
D = 64
N = B * S

WIN = 512


def _emb_body(table_tile, idx_vmem, out_vmem):
    pltpu.sync_copy(table_tile.at[idx_vmem.at[0]], out_vmem)


def _kernel_body(hour_hbm, week_hbm, wh_hbm, ww_hbm, week_out, hour_out):
    for table, idx, out in ((wh_hbm, hour_hbm, hour_out),
                            (ww_hbm, week_hbm, week_out)):
        pltpu.emit_pipeline(
            functools.partial(_emb_body, table),
            grid=(N // WIN,),
            in_specs=[pl.BlockSpec((1, WIN), index_map=lambda i: (0, i))],
            out_specs=[pl.BlockSpec((WIN, D), index_map=lambda i: (i, 0))],
            core_axis_name=("core", "subcore"),
            dimension_semantics=(pltpu.PARALLEL,),
        )(idx, out)


def kernel(hour, week, W_hour, W_week):
    mesh = plsc.VectorSubcoreMesh(core_axis_name="core",
                                  subcore_axis_name="subcore")
    out_t = jax.ShapeDtypeStruct((N, D), jnp.float32)
    k = pl.kernel(_kernel_body, mesh=mesh, out_type=(out_t, out_t),
                  compiler_params=pltpu.CompilerParams(
                      use_tc_tiling_on_sc=False))
    week_e, hour_e = k(hour.reshape(1, N), week.reshape(1, N), W_hour, W_week)
    return (week_e.reshape(B, S, D), hour_e.reshape(B, S, D))


# speedup vs baseline: 5.9318x; 5.9318x over previous
import functools

import jax
import jax.numpy as jnp
from jax import lax
from jax.experimental import pallas as pl
from jax.experimental.pallas import tpu as pltpu
from jax.experimental.pallas import tpu_sc as plsc

B = 16384
S = 50
D = 64
N = B * S

WIN = 512


def _emb_body(table_tile, idx_vmem, out_vmem):
    pltpu.sync_copy(table_tile.at[idx_vmem.at[0]], out_vmem)


def _kernel_body(hour_hbm, week_hbm, wh_hbm, ww_hbm, week_out, hour_out,
                 wh_sp, ww_sp):
    sid = lax.axis_index("subcore")

    @pl.when(sid == 0)
    def _stage():
        pltpu.sync_copy(wh_hbm, wh_sp)
        pltpu.sync_copy(ww_hbm, ww_sp)

    plsc.subcore_barrier()

    for table, idx, out in ((wh_sp, hour_hbm, hour_out),
                            (ww_sp, week_hbm, week_out)):
        pltpu.emit_pipeline(
            functools.partial(_emb_body, table),
            grid=(N // WIN,),
            in_specs=[pl.BlockSpec((1, WIN), index_map=lambda i: (0, i))],
            out_specs=[pl.BlockSpec((WIN, D), index_map=lambda i: (i, 0))],
            core_axis_name=("core", "subcore"),
            dimension_semantics=(pltpu.PARALLEL,),
        )(idx, out)


def kernel(hour, week, W_hour, W_week):
    mesh = plsc.VectorSubcoreMesh(core_axis_name="core",
                                  subcore_axis_name="subcore")
    out_t = jax.ShapeDtypeStruct((N, D), jnp.float32)
    k = pl.kernel(_kernel_body, mesh=mesh, out_type=(out_t, out_t),
                  scratch_types=[
                      pltpu.VMEM_SHARED((25, D), jnp.float32),
                      pltpu.VMEM_SHARED((8, D), jnp.float32),
                  ],
                  compiler_params=pltpu.CompilerParams(
                      use_tc_tiling_on_sc=False))
    week_e, hour_e = k(hour.reshape(1, N), week.reshape(1, N), W_hour, W_week)
    return (week_e.reshape(B, S, D), hour_e.reshape(B, S, D))


# speedup vs baseline: 12.5562x; 2.1168x over previous
import functools

import jax
import jax.numpy as jnp
from jax import lax
from jax.experimental import pallas as pl
from jax.experimental.pallas import tpu as pltpu
from jax.experimental.pallas import tpu_sc as plsc

B = 16384
S = 50
D = 64
N = B * S

WIN = 512


def _emb_body(table_tile, idx_vmem, out_vmem):
    pltpu.sync_copy(table_tile.at[idx_vmem.at[0]], out_vmem)


def _kernel_body(hour_hbm, week_hbm, wh_hbm, ww_hbm, week_out, hour_out,
                 wh_sp, ww_sp):
    sid = lax.axis_index("subcore")

    @pl.when(sid == 0)
    def _stage():
        pltpu.sync_copy(wh_hbm, wh_sp)
        pltpu.sync_copy(ww_hbm, ww_sp)

    plsc.subcore_barrier()

    for table, idx, out in ((wh_sp, hour_hbm, hour_out),
                            (ww_sp, week_hbm, week_out)):
        pltpu.emit_pipeline(
            functools.partial(_emb_body, table),
            grid=(N // WIN,),
            in_specs=[pl.BlockSpec((1, WIN), index_map=lambda i: (0, i))],
            out_specs=[pl.BlockSpec((WIN, D), index_map=lambda i: (i, 0))],
            core_axis_name=("core", "subcore"),
            dimension_semantics=(pltpu.PARALLEL,),
        )(idx, out)


BC = 256


def _tr_body(x_ref, o_ref):
    o_ref[...] = x_ref[...].reshape(BC, S * D).T


def _format(y):
    y2 = y.reshape(N // 2, 2 * D)
    p2 = pl.pallas_call(
        _tr_body,
        grid=(B // BC,),
        in_specs=[pl.BlockSpec((BC * S // 2, 2 * D), lambda i: (i, 0))],
        out_specs=pl.BlockSpec((S * D, BC), lambda i: (0, i)),
        out_shape=jax.ShapeDtypeStruct((S * D, B), jnp.float32),
    )(y2)
    return p2.reshape(S, D, B).transpose(2, 0, 1)


def kernel(hour, week, W_hour, W_week):
    mesh = plsc.VectorSubcoreMesh(core_axis_name="core",
                                  subcore_axis_name="subcore")
    out_t = jax.ShapeDtypeStruct((N, D), jnp.float32)
    k = pl.kernel(_kernel_body, mesh=mesh, out_type=(out_t, out_t),
                  scratch_types=[
                      pltpu.VMEM_SHARED((25, D), jnp.float32),
                      pltpu.VMEM_SHARED((8, D), jnp.float32),
                  ],
                  compiler_params=pltpu.CompilerParams(
                      use_tc_tiling_on_sc=False))
    week_e, hour_e = k(hour.reshape(1, N), week.reshape(1, N), W_hour, W_week)
    return (_format(week_e), _format(hour_e))


# speedup vs baseline: 13.8015x; 1.0992x over previous
import functools

import jax
import jax.numpy as jnp
from jax import lax
from jax.experimental import pallas as pl
from jax.experimental.pallas import tpu as pltpu
from jax.experimental.pallas import tpu_sc as plsc

B = 16384
S = 50
D = 64
N = B * S

WIN = 512


def _emb_body(table_tile, idx_vmem, out_vmem):
    pltpu.sync_copy(table_tile.at[idx_vmem.at[0]], out_vmem)


def _kernel_body(idx_hbm, w_hbm, out, w_sp):
    sid = lax.axis_index("subcore")

    @pl.when(sid == 0)
    def _stage():
        pltpu.sync_copy(w_hbm, w_sp)

    plsc.subcore_barrier()

    pltpu.emit_pipeline(
        functools.partial(_emb_body, w_sp),
        grid=(N // WIN,),
        in_specs=[pl.BlockSpec((1, WIN), index_map=lambda i: (0, i))],
        out_specs=[pl.BlockSpec((WIN, D), index_map=lambda i: (i, 0))],
        core_axis_name=("core", "subcore"),
        dimension_semantics=(pltpu.PARALLEL,),
    )(idx_hbm, out)


def _gather(idx, W):
    mesh = plsc.VectorSubcoreMesh(core_axis_name="core",
                                  subcore_axis_name="subcore")
    k = pl.kernel(_kernel_body, mesh=mesh,
                  out_type=jax.ShapeDtypeStruct((N, D), jnp.float32),
                  scratch_types=[pltpu.VMEM_SHARED(W.shape, jnp.float32)],
                  compiler_params=pltpu.CompilerParams(
                      use_tc_tiling_on_sc=False))
    return k(idx.reshape(1, N), W)


BC = 256


def _tr_body(x_ref, o_ref):
    o_ref[...] = x_ref[...].reshape(BC, S * D).T


def _format(y):
    y2 = y.reshape(N // 2, 2 * D)
    p2 = pl.pallas_call(
        _tr_body,
        grid=(B // BC,),
        in_specs=[pl.BlockSpec((BC * S // 2, 2 * D), lambda i: (i, 0))],
        out_specs=pl.BlockSpec((S * D, BC), lambda i: (0, i)),
        out_shape=jax.ShapeDtypeStruct((S * D, B), jnp.float32),
    )(y2)
    return p2.reshape(S, D, B).transpose(2, 0, 1)


def kernel(hour, week, W_hour, W_week):
    week_e = _gather(week, W_week)
    hour_e = _gather(hour, W_hour)
    return (_format(week_e), _format(hour_e))
